# Initial kernel scaffold; baseline (speedup 1.0000x reference)
import jax
import jax.numpy as jnp
from jax.experimental import pallas as pl


def kernel(x, pos_embedding):
    raise NotImplementedError("write your pallas kernel here")


# speedup vs baseline: 3.6314x; 3.6314x over previous
import functools

import jax
import jax.numpy as jnp
from jax import lax
from jax.experimental import pallas as pl
from jax.experimental.pallas import tpu as pltpu
from jax.experimental.pallas import tpu_sc as plsc


def _broadcast_rows(table, batch, chunk_rows):
    S, D = table.shape
    info = plsc.get_sparse_core_info()
    nw = info.num_cores * info.num_subcores
    rows_per_w = S // nw
    n_ch = rows_per_w // chunk_rows
    mesh = plsc.VectorSubcoreMesh(core_axis_name="c", subcore_axis_name="s")

    @functools.partial(
        pl.kernel,
        mesh=mesh,
        out_type=jax.ShapeDtypeStruct((batch * S, D), table.dtype),
        scratch_types=[pltpu.VMEM((chunk_rows, D), table.dtype)],
    )
    def k(table_hbm, out_hbm, buf):
        wid = lax.axis_index("s") * info.num_cores + lax.axis_index("c")
        base = wid * rows_per_w
        for c in range(n_ch):
            r0 = base + c * chunk_rows
            pltpu.sync_copy(table_hbm.at[pl.ds(r0, chunk_rows), :], buf)
            for b in range(batch):
                pltpu.sync_copy(buf, out_hbm.at[pl.ds(b * S + r0, chunk_rows), :])

    return k(table)


def kernel(x, pos_embedding):
    batch, seq = x.shape
    table = pos_embedding[:seq]
    out = _broadcast_rows(table, batch, chunk_rows=64)
    return out.reshape(batch, seq, pos_embedding.shape[1])
